# baseline (speedup 1.0000x reference)
import jax
import jax.numpy as jnp
from jax import lax
from jax.experimental import pallas as pl
from jax.experimental.pallas import tpu as pltpu

_UNK = 22
_THRESH = 838861
_K0 = 0
_K1 = 42
_KS2 = _K0 ^ _K1 ^ 0x1BD11BDA
_ROT = ((13, 15, 26, 6), (17, 29, 16, 24))


def _rotl(v, d):
    return lax.shift_right_logical(v, jnp.int32(32 - d)) | (v << jnp.int32(d))


def _threefry_bits_prekeyed(x1):
    x0 = x1
    x1 = _rotl(x1, _ROT[0][0]) ^ x0
    for d in _ROT[0][1:]:
        x0 = x0 + x1
        x1 = _rotl(x1, d) ^ x0
    ks = (_K0, _K1, _KS2)
    x0 = x0 + jnp.int32(ks[1])
    x1 = x1 + jnp.int32(ks[2] + 1)
    for i in range(1, 5):
        for d in _ROT[i % 2]:
            x0 = x0 + x1
            x1 = _rotl(x1, d) ^ x0
        x0 = x0 + jnp.int32(ks[(i + 1) % 3])
        x1 = x1 + jnp.int32(ks[(i + 2) % 3] + i + 1)
    return x0 ^ x1


def _packed_mask(rows, seg_cols, total_cols, col_base):
    half = seg_cols // 2
    s = lax.broadcasted_iota(jnp.int32, (2 * rows, 1), 0)
    row_off = (s & jnp.int32(rows - 1)) * jnp.int32(total_cols) + (
        lax.shift_right_logical(s, jnp.int32(rows.bit_length() - 1))
        * jnp.int32(half)
    ) + jnp.int32(col_base + _K1)
    l = lax.broadcasted_iota(jnp.int32, (2 * rows, half), 1)
    bits = _threefry_bits_prekeyed(row_off + l)
    m8 = lax.shift_right_logical(bits, jnp.int32(9)) < jnp.int32(_THRESH)
    return jnp.concatenate([m8[:rows, :], m8[rows:, :]], axis=1)


def _threefry_mask_body(x_hbm, o_hbm, xv, ov, si0, so0, so1):
    rows, cols = xv.shape
    half = cols // 2
    cin = pltpu.make_async_copy(x_hbm, xv, si0)
    cin.start()
    mask0 = _packed_mask(rows, half, cols, 0)
    cin.wait()
    sl0 = (slice(None), pl.ds(0, half))
    sl1 = (slice(None), pl.ds(half, half))
    ov[sl0] = jnp.where(mask0, jnp.int32(_UNK), xv[sl0])
    out0 = pltpu.make_async_copy(ov.at[sl0], o_hbm.at[sl0], so0)
    out0.start()
    mask1 = _packed_mask(rows, half, cols, half)
    ov[sl1] = jnp.where(mask1, jnp.int32(_UNK), xv[sl1])
    out1 = pltpu.make_async_copy(ov.at[sl1], o_hbm.at[sl1], so1)
    out1.start()
    out0.wait()
    out1.wait()


@jax.jit
def kernel(x):
    return pl.pallas_call(
        _threefry_mask_body,
        in_specs=[pl.BlockSpec(memory_space=pltpu.MemorySpace.HBM)],
        out_specs=pl.BlockSpec(memory_space=pltpu.MemorySpace.HBM),
        out_shape=jax.ShapeDtypeStruct(x.shape, x.dtype),
        scratch_shapes=[
            pltpu.VMEM(x.shape, jnp.int32),
            pltpu.VMEM(x.shape, jnp.int32),
            pltpu.SemaphoreType.DMA,
            pltpu.SemaphoreType.DMA,
            pltpu.SemaphoreType.DMA,
        ],
    )(x)


# speedup vs baseline: 1.0151x; 1.0151x over previous
import jax
import jax.numpy as jnp
from jax import lax
from jax.experimental import pallas as pl
from jax.experimental.pallas import tpu as pltpu

_UNK = 22
_THRESH = 838861
_K0 = 0
_K1 = 42
_KS2 = _K0 ^ _K1 ^ 0x1BD11BDA
_ROT = ((13, 15, 26, 6), (17, 29, 16, 24))


def _rotl(v, d):
    return lax.shift_right_logical(v, jnp.int32(32 - d)) | (v << jnp.int32(d))


def _threefry_bits_prekeyed(x1):
    x0 = x1
    x1 = _rotl(x1, _ROT[0][0]) ^ x0
    for d in _ROT[0][1:]:
        x0 = x0 + x1
        x1 = _rotl(x1, d) ^ x0
    ks = (_K0, _K1, _KS2)
    x0 = x0 + jnp.int32(ks[1])
    x1 = x1 + jnp.int32(ks[2] + 1)
    for i in range(1, 5):
        for d in _ROT[i % 2]:
            x0 = x0 + x1
            x1 = _rotl(x1, d) ^ x0
        x0 = x0 + jnp.int32(ks[(i + 1) % 3])
        x1 = x1 + jnp.int32(ks[(i + 2) % 3] + i + 1)
    return x0 ^ x1


def _packed_mask(rows, seg_cols, total_cols, col_base):
    half = seg_cols // 2
    s = lax.broadcasted_iota(jnp.int32, (2 * rows, 1), 0)
    row_off = (s & jnp.int32(rows - 1)) * jnp.int32(total_cols) + (
        lax.shift_right_logical(s, jnp.int32(rows.bit_length() - 1))
        * jnp.int32(half)
    ) + (jnp.int32(_K1) + col_base)
    l = lax.broadcasted_iota(jnp.int32, (2 * rows, half), 1)
    bits = _threefry_bits_prekeyed(row_off + l)
    m8 = lax.shift_right_logical(bits, jnp.int32(9)) < jnp.int32(_THRESH)
    return jnp.concatenate([m8[:rows, :], m8[rows:, :]], axis=1)


def _threefry_mask_body(x_hbm, o_ref, xv, sem):
    rows, cols = xv.shape
    half = cols // 2
    i = pl.program_id(0)
    copy = pltpu.make_async_copy(x_hbm, xv, sem)

    @pl.when(i == 0)
    def _():
        copy.start()

    mask = _packed_mask(rows, half, cols, i * jnp.int32(half))

    @pl.when(i == 0)
    def _():
        copy.wait()

    o_ref[...] = jnp.where(
        mask, jnp.int32(_UNK), xv[:, pl.ds(i * jnp.int32(half), half)]
    )


@jax.jit
def kernel(x):
    rows, cols = x.shape
    half = cols // 2
    return pl.pallas_call(
        _threefry_mask_body,
        grid=(2,),
        in_specs=[pl.BlockSpec(memory_space=pltpu.MemorySpace.HBM)],
        out_specs=pl.BlockSpec((rows, half), lambda i: (0, i)),
        out_shape=jax.ShapeDtypeStruct(x.shape, x.dtype),
        scratch_shapes=[
            pltpu.VMEM(x.shape, jnp.int32),
            pltpu.SemaphoreType.DMA,
        ],
    )(x)


# speedup vs baseline: 1.1549x; 1.1377x over previous
import jax
import jax.numpy as jnp
from jax import lax
from jax.experimental import pallas as pl
from jax.experimental.pallas import tpu as pltpu

_UNK = 22
_THRESH = 838861
_K0 = 0
_K1 = 42
_KS2 = _K0 ^ _K1 ^ 0x1BD11BDA
_ROT = ((13, 15, 26, 6), (17, 29, 16, 24))


def _rotl(v, d):
    return lax.shift_right_logical(v, jnp.int32(32 - d)) | (v << jnp.int32(d))


def _threefry_bits_prekeyed(x1):
    x0 = x1
    x1 = _rotl(x1, _ROT[0][0]) ^ x0
    for d in _ROT[0][1:]:
        x0 = x0 + x1
        x1 = _rotl(x1, d) ^ x0
    ks = (_K0, _K1, _KS2)
    x0 = x0 + jnp.int32(ks[1])
    x1 = x1 + jnp.int32(ks[2] + 1)
    for i in range(1, 5):
        for d in _ROT[i % 2]:
            x0 = x0 + x1
            x1 = _rotl(x1, d) ^ x0
        x0 = x0 + jnp.int32(ks[(i + 1) % 3])
        x1 = x1 + jnp.int32(ks[(i + 2) % 3] + i + 1)
    return x0 ^ x1


def _packed_mask(rows, seg_cols, total_cols, col_base):
    half = seg_cols // 2
    s = lax.broadcasted_iota(jnp.int32, (2 * rows, 1), 0)
    row_off = (s & jnp.int32(rows - 1)) * jnp.int32(total_cols) + (
        lax.shift_right_logical(s, jnp.int32(rows.bit_length() - 1))
        * jnp.int32(half)
    ) + jnp.int32(col_base + _K1)
    l = lax.broadcasted_iota(jnp.int32, (2 * rows, half), 1)
    bits = _threefry_bits_prekeyed(row_off + l)
    m8 = lax.shift_right_logical(bits, jnp.int32(9)) < jnp.int32(_THRESH)
    return jnp.concatenate([m8[:rows, :], m8[rows:, :]], axis=1)


def _threefry_mask_body(x_hbm, o_ref, xv, sem):
    rows, cols = xv.shape
    copy = pltpu.make_async_copy(x_hbm, xv, sem)
    copy.start()
    mask = _packed_mask(rows, cols, cols, 0)
    copy.wait()
    o_ref[...] = jnp.where(mask, jnp.int32(_UNK), xv[...])


@jax.jit
def kernel(x):
    return pl.pallas_call(
        _threefry_mask_body,
        in_specs=[pl.BlockSpec(memory_space=pltpu.MemorySpace.HBM)],
        out_shape=jax.ShapeDtypeStruct(x.shape, x.dtype),
        scratch_shapes=[
            pltpu.VMEM(x.shape, jnp.int32),
            pltpu.SemaphoreType.DMA,
        ],
    )(x)


# speedup vs baseline: 1.1599x; 1.0043x over previous
import jax
import jax.numpy as jnp
from jax import lax
from jax.experimental import pallas as pl
from jax.experimental.pallas import tpu as pltpu

_UNK = 22
_THRESH = 838861
_K0 = 0
_K1 = 42
_KS2 = _K0 ^ _K1 ^ 0x1BD11BDA
_ROT = ((13, 15, 26, 6), (17, 29, 16, 24))


def _rotl(v, d):
    return lax.shift_right_logical(v, jnp.int32(32 - d)) | (v << jnp.int32(d))


def _threefry_bits_prekeyed(x1):
    x0 = x1
    x1 = _rotl(x1, _ROT[0][0]) ^ x0
    for d in _ROT[0][1:]:
        x0 = x0 + x1
        x1 = _rotl(x1, d) ^ x0
    ks = (_K0, _K1, _KS2)
    x0 = x0 + jnp.int32(ks[1])
    x1 = x1 + jnp.int32(ks[2] + 1)
    for i in range(1, 5):
        for d in _ROT[i % 2]:
            x0 = x0 + x1
            x1 = _rotl(x1, d) ^ x0
        x0 = x0 + jnp.int32(ks[(i + 1) % 3])
        x1 = x1 + jnp.int32(ks[(i + 2) % 3] + i + 1)
    return x0 ^ x1


def _packed_mask(rows, seg_cols, total_cols, col_base):
    half = seg_cols // 2
    s = lax.broadcasted_iota(jnp.int32, (2 * rows, 1), 0)
    row_off = (s & jnp.int32(rows - 1)) * jnp.int32(total_cols) + (
        lax.shift_right_logical(s, jnp.int32(rows.bit_length() - 1))
        * jnp.int32(half)
    ) + jnp.int32(col_base + _K1)
    l = lax.broadcasted_iota(jnp.int32, (2 * rows, half), 1)
    bits = _threefry_bits_prekeyed(row_off + l)
    return lax.shift_right_logical(bits, jnp.int32(9)) < jnp.int32(_THRESH)


def _threefry_mask_body(x_hbm, o_ref, xv, sem):
    rows, cols = xv.shape
    half = cols // 2
    copy = pltpu.make_async_copy(x_hbm, xv, sem)
    copy.start()
    m8 = _packed_mask(rows, cols, cols, 0)
    copy.wait()
    unk = jnp.int32(_UNK)
    o_ref[:, :half] = jnp.where(m8[:rows, :], unk, xv[:, :half])
    o_ref[:, half:] = jnp.where(m8[rows:, :], unk, xv[:, half:])


@jax.jit
def kernel(x):
    return pl.pallas_call(
        _threefry_mask_body,
        in_specs=[pl.BlockSpec(memory_space=pltpu.MemorySpace.HBM)],
        out_shape=jax.ShapeDtypeStruct(x.shape, x.dtype),
        scratch_shapes=[
            pltpu.VMEM(x.shape, jnp.int32),
            pltpu.SemaphoreType.DMA,
        ],
    )(x)
